# baseline (device time: 202548 ns/iter reference)
import functools

import jax
import jax.numpy as jnp
from jax import lax
from jax.experimental import pallas as pl
from jax.experimental.pallas import tpu as pltpu

N_DEV = 32
N_EXPERTS = 128
CAPACITY = 102.0
F_HOPS = 16
B_HOPS = 15
SLOTS_F = 9
SLOTS_B = 8
CHUNKS = 4

_POS = {}
_p = 0
for _z in range(4):
    for _y in range(4):
        for _x in ([0, 1] if _y % 2 == 0 else [1, 0]):
            _POS[(_x, _y, _z)] = _p
            _p += 1

_SNAKE_YZ = [(0, 0), (1, 0), (2, 0), (3, 0), (3, 1), (2, 1), (1, 1), (0, 1),
             (0, 2), (1, 2), (2, 2), (3, 2), (3, 3), (2, 3), (1, 3), (0, 3)]
_CYC = ([(0, y, z) for (y, z) in _SNAKE_YZ]
        + [(1, y, z) for (y, z) in reversed(_SNAKE_YZ)])
RING = [_POS[c] for c in _CYC]
SIGMA = [RING.index(m) for m in range(N_DEV)]


def kernel(x, router_W, route_idx, expert_W):
    del router_W
    tokens, d = x.shape
    e_loc, _, h = expert_W.shape

    my = lax.axis_index("i")
    ring = jnp.asarray(RING, jnp.int32)
    sigma = jnp.asarray(SIGMA, jnp.int32)
    ci = sigma[my]
    nxt = ring[(ci + 1) % N_DEV]
    prv = ring[(ci - 1) % N_DEV]
    ofs = ring[(ci - jnp.arange(1, F_HOPS + 1)) % N_DEV]
    obs = ring[(ci + jnp.arange(1, B_HOPS + 1)) % N_DEV]
    meta = jnp.concatenate(
        [jnp.stack([nxt, prv]), ofs, obs]).astype(jnp.int32).reshape(1, -1)

    def body(x_ref, ridx_ref, ew_ref, meta_ref, out_ref,
             comm_f, comm_b, comm_cf, comm_cb,
             sw_f, rw_f, sw_b, rw_b, sc_f, rc_f, sc_b, rc_b):
        my = lax.axis_index("i")
        right = meta_ref[0, 0]
        left = meta_ref[0, 1]

        def origin_f(hop):
            return meta_ref[0, 2 + (hop - 1)]

        def origin_b(hop):
            return meta_ref[0, 2 + F_HOPS + (hop - 1)]

        barrier = pltpu.get_barrier_semaphore()
        for nbr in (left, right):
            pl.semaphore_signal(barrier, inc=1, device_id=(nbr,),
                                device_id_type=pl.DeviceIdType.MESH)
        pl.semaphore_wait(barrier, 2)

        own_w = ew_ref[:, :, :].astype(jnp.bfloat16)
        comm_f[0, :, :, :] = own_w
        comm_b[0, :, :, :] = own_w

        def accum(origin, comm, slot):
            for j in range(e_loc):
                e = origin * e_loc + j
                m = (route == e).astype(jnp.bfloat16)
                out_ref[:, :] += jnp.dot(
                    x_bf * m, comm[slot, j, :, :],
                    preferred_element_type=jnp.float32)

        def pfx(prefix, origin, comm_c, slot):
            cnts = comm_c[slot, 0:1, :].astype(jnp.float32)
            return prefix + jnp.where(origin < my, cnts, 0.0)

        epc = e_loc // CHUNKS

        def mk(comm, comm_c, ssem, rsem, csem, crsem, dst, hop, n_slots):
            rws = [pltpu.make_async_remote_copy(
                src_ref=comm.at[(hop - 1) % n_slots, pl.ds(k * epc, epc)],
                dst_ref=comm.at[hop % n_slots, pl.ds(k * epc, epc)],
                send_sem=ssem.at[(hop - 1) * CHUNKS + k],
                recv_sem=rsem.at[(hop - 1) * CHUNKS + k],
                device_id=(dst,), device_id_type=pl.DeviceIdType.MESH)
                for k in range(CHUNKS)]
            rc = pltpu.make_async_remote_copy(
                src_ref=comm_c.at[hop - 1], dst_ref=comm_c.at[hop],
                send_sem=csem.at[hop - 1], recv_sem=crsem.at[hop - 1],
                device_id=(dst,), device_id_type=pl.DeviceIdType.MESH)
            return rws + [rc]

        fwd = [mk(comm_f, comm_cf, sw_f, rw_f, sc_f, rc_f, right, hp, SLOTS_F)
               for hp in range(1, F_HOPS + 1)]
        bwd = [mk(comm_b, comm_cb, sw_b, rw_b, sc_b, rc_b, left, hp, SLOTS_B)
               for hp in range(1, B_HOPS + 1)]

        for r_ in fwd[0][:CHUNKS] + bwd[0][:CHUNKS]:
            r_.start()

        x_bf = x_ref[:, :].astype(jnp.bfloat16)
        route = ridx_ref[:, :]

        eids = lax.broadcasted_iota(jnp.int32, (tokens, N_EXPERTS), 1)
        oh = (route == eids).astype(jnp.float32)
        row = lax.broadcasted_iota(jnp.int32, (tokens, tokens), 0)
        col = lax.broadcasted_iota(jnp.int32, (tokens, tokens), 1)
        lower = (col < row).astype(jnp.float32)
        csum_excl = jnp.dot(lower, oh, preferred_element_type=jnp.float32)
        rank = jnp.sum(csum_excl * oh, axis=1, keepdims=True)
        counts = jnp.sum(oh, axis=0, keepdims=True)

        cbcast = jnp.broadcast_to(counts.astype(jnp.int32), comm_cf.shape[1:])
        comm_cf[0, :, :] = cbcast
        comm_cb[0, :, :] = cbcast
        fwd[0][CHUNKS].start()
        bwd[0][CHUNKS].start()

        out_ref[:, :] = jnp.zeros((tokens, h), jnp.float32)
        accum(my, comm_f, 0)

        prefix = jnp.zeros((1, N_EXPERTS), jnp.float32)
        for hop in range(1, F_HOPS + 1):
            n_pieces = CHUNKS + 1
            for k in range(n_pieces):
                fwd[hop - 1][k].wait_recv()
                if hop < F_HOPS:
                    fwd[hop][k].start()
                if hop <= B_HOPS:
                    bwd[hop - 1][k].wait_recv()
                    if hop < B_HOPS:
                        bwd[hop][k].start()

            of = origin_f(hop)
            accum(of, comm_f, hop % SLOTS_F)
            prefix = pfx(prefix, of, comm_cf, hop)
            if hop <= B_HOPS:
                ob = origin_b(hop)
                accum(ob, comm_b, hop % SLOTS_B)
                prefix = pfx(prefix, ob, comm_cb, hop)

        my_prefix = jnp.sum(oh * prefix, axis=1, keepdims=True)
        accept = ((my_prefix + rank) < CAPACITY).astype(jnp.float32)
        out_ref[:, :] *= accept

        for group in fwd + bwd:
            for r_ in group:
                r_.wait_send()

        @functools.partial(pl.run_scoped,
                           second_barrier=pltpu.SemaphoreType.REGULAR)
        def _(second_barrier):
            for nbr in (left, right):
                pl.semaphore_signal(second_barrier, inc=1, device_id=(nbr,),
                                    device_id_type=pl.DeviceIdType.MESH)
            pl.semaphore_wait(second_barrier, 2)

    return pl.pallas_call(
        body,
        out_shape=jax.ShapeDtypeStruct((tokens, h), jnp.float32),
        in_specs=[pl.BlockSpec(memory_space=pltpu.VMEM)] * 3
        + [pl.BlockSpec(memory_space=pltpu.SMEM)],
        out_specs=pl.BlockSpec(memory_space=pltpu.VMEM),
        scratch_shapes=[
            pltpu.VMEM((SLOTS_F, e_loc, d, h), jnp.bfloat16),
            pltpu.VMEM((SLOTS_B, e_loc, d, h), jnp.bfloat16),
            pltpu.VMEM((F_HOPS + 1, 8, N_EXPERTS), jnp.int32),
            pltpu.VMEM((B_HOPS + 1, 8, N_EXPERTS), jnp.int32),
            pltpu.SemaphoreType.DMA((F_HOPS * CHUNKS,)),
            pltpu.SemaphoreType.DMA((F_HOPS * CHUNKS,)),
            pltpu.SemaphoreType.DMA((B_HOPS * CHUNKS,)),
            pltpu.SemaphoreType.DMA((B_HOPS * CHUNKS,)),
            pltpu.SemaphoreType.DMA((F_HOPS,)),
            pltpu.SemaphoreType.DMA((F_HOPS,)),
            pltpu.SemaphoreType.DMA((B_HOPS,)),
            pltpu.SemaphoreType.DMA((B_HOPS,)),
        ],
        compiler_params=pltpu.CompilerParams(collective_id=0),
    )(x, route_idx, expert_W, meta)


# device time: 201240 ns/iter; 1.0065x vs baseline; 1.0065x over previous
import functools

import jax
import jax.numpy as jnp
from jax import lax
from jax.experimental import pallas as pl
from jax.experimental.pallas import tpu as pltpu

N_DEV = 32
N_EXPERTS = 128
CAPACITY = 102.0
F_HOPS = 16
B_HOPS = 15
SLOTS_F = 9
SLOTS_B = 8
CHUNKS = 2

_POS = {}
_p = 0
for _z in range(4):
    for _y in range(4):
        for _x in ([0, 1] if _y % 2 == 0 else [1, 0]):
            _POS[(_x, _y, _z)] = _p
            _p += 1

_SNAKE_YZ = [(0, 0), (1, 0), (2, 0), (3, 0), (3, 1), (2, 1), (1, 1), (0, 1),
             (0, 2), (1, 2), (2, 2), (3, 2), (3, 3), (2, 3), (1, 3), (0, 3)]
_CYC = ([(0, y, z) for (y, z) in _SNAKE_YZ]
        + [(1, y, z) for (y, z) in reversed(_SNAKE_YZ)])
RING = [_POS[c] for c in _CYC]
SIGMA = [RING.index(m) for m in range(N_DEV)]


def kernel(x, router_W, route_idx, expert_W):
    del router_W
    tokens, d = x.shape
    e_loc, _, h = expert_W.shape

    my = lax.axis_index("i")
    ring = jnp.asarray(RING, jnp.int32)
    sigma = jnp.asarray(SIGMA, jnp.int32)
    ci = sigma[my]
    nxt = ring[(ci + 1) % N_DEV]
    prv = ring[(ci - 1) % N_DEV]
    ofs = ring[(ci - jnp.arange(1, F_HOPS + 1)) % N_DEV]
    obs = ring[(ci + jnp.arange(1, B_HOPS + 1)) % N_DEV]
    meta = jnp.concatenate(
        [jnp.stack([nxt, prv]), ofs, obs]).astype(jnp.int32).reshape(1, -1)

    def body(x_ref, ridx_ref, ew_ref, meta_ref, out_ref,
             comm_f, comm_b, comm_cf, comm_cb,
             sw_f, rw_f, sw_b, rw_b, sc_f, rc_f, sc_b, rc_b):
        my = lax.axis_index("i")
        right = meta_ref[0, 0]
        left = meta_ref[0, 1]

        def origin_f(hop):
            return meta_ref[0, 2 + (hop - 1)]

        def origin_b(hop):
            return meta_ref[0, 2 + F_HOPS + (hop - 1)]

        barrier = pltpu.get_barrier_semaphore()
        for nbr in (left, right):
            pl.semaphore_signal(barrier, inc=1, device_id=(nbr,),
                                device_id_type=pl.DeviceIdType.MESH)
        pl.semaphore_wait(barrier, 2)

        own_w = ew_ref[:, :, :].astype(jnp.bfloat16)
        comm_f[0, :, :, :] = own_w
        comm_b[0, :, :, :] = own_w

        def accum(origin, comm, slot):
            for j in range(e_loc):
                e = origin * e_loc + j
                m = (route == e).astype(jnp.bfloat16)
                out_ref[:, :] += jnp.dot(
                    x_bf * m, comm[slot, j, :, :],
                    preferred_element_type=jnp.float32)

        def pfx(prefix, origin, comm_c, slot):
            cnts = comm_c[slot, 0:1, :].astype(jnp.float32)
            return prefix + jnp.where(origin < my, cnts, 0.0)

        epc = e_loc // CHUNKS

        def mk(comm, comm_c, ssem, rsem, csem, crsem, dst, hop, n_slots):
            rws = [pltpu.make_async_remote_copy(
                src_ref=comm.at[(hop - 1) % n_slots, pl.ds(k * epc, epc)],
                dst_ref=comm.at[hop % n_slots, pl.ds(k * epc, epc)],
                send_sem=ssem.at[(hop - 1) * CHUNKS + k],
                recv_sem=rsem.at[(hop - 1) * CHUNKS + k],
                device_id=(dst,), device_id_type=pl.DeviceIdType.MESH)
                for k in range(CHUNKS)]
            rc = pltpu.make_async_remote_copy(
                src_ref=comm_c.at[hop - 1], dst_ref=comm_c.at[hop],
                send_sem=csem.at[hop - 1], recv_sem=crsem.at[hop - 1],
                device_id=(dst,), device_id_type=pl.DeviceIdType.MESH)
            return rws + [rc]

        fwd = [mk(comm_f, comm_cf, sw_f, rw_f, sc_f, rc_f, right, hp, SLOTS_F)
               for hp in range(1, F_HOPS + 1)]
        bwd = [mk(comm_b, comm_cb, sw_b, rw_b, sc_b, rc_b, left, hp, SLOTS_B)
               for hp in range(1, B_HOPS + 1)]

        for r_ in fwd[0][:CHUNKS] + bwd[0][:CHUNKS]:
            r_.start()

        x_bf = x_ref[:, :].astype(jnp.bfloat16)
        route = ridx_ref[:, :]

        eids = lax.broadcasted_iota(jnp.int32, (tokens, N_EXPERTS), 1)
        oh = (route == eids).astype(jnp.float32)
        row = lax.broadcasted_iota(jnp.int32, (tokens, tokens), 0)
        col = lax.broadcasted_iota(jnp.int32, (tokens, tokens), 1)
        lower = (col < row).astype(jnp.float32)
        csum_excl = jnp.dot(lower, oh, preferred_element_type=jnp.float32)
        rank = jnp.sum(csum_excl * oh, axis=1, keepdims=True)
        counts = jnp.sum(oh, axis=0, keepdims=True)

        cbcast = jnp.broadcast_to(counts.astype(jnp.int32), comm_cf.shape[1:])
        comm_cf[0, :, :] = cbcast
        comm_cb[0, :, :] = cbcast
        fwd[0][CHUNKS].start()
        bwd[0][CHUNKS].start()

        out_ref[:, :] = jnp.zeros((tokens, h), jnp.float32)
        accum(my, comm_f, 0)

        prefix = jnp.zeros((1, N_EXPERTS), jnp.float32)
        for hop in range(1, F_HOPS + 1):
            n_pieces = CHUNKS + 1
            for k in range(n_pieces):
                fwd[hop - 1][k].wait_recv()
                if hop < F_HOPS:
                    fwd[hop][k].start()
                if hop <= B_HOPS:
                    bwd[hop - 1][k].wait_recv()
                    if hop < B_HOPS:
                        bwd[hop][k].start()

            of = origin_f(hop)
            accum(of, comm_f, hop % SLOTS_F)
            prefix = pfx(prefix, of, comm_cf, hop)
            if hop <= B_HOPS:
                ob = origin_b(hop)
                accum(ob, comm_b, hop % SLOTS_B)
                prefix = pfx(prefix, ob, comm_cb, hop)

        my_prefix = jnp.sum(oh * prefix, axis=1, keepdims=True)
        accept = ((my_prefix + rank) < CAPACITY).astype(jnp.float32)
        out_ref[:, :] *= accept

        for group in fwd + bwd:
            for r_ in group:
                r_.wait_send()

        @functools.partial(pl.run_scoped,
                           second_barrier=pltpu.SemaphoreType.REGULAR)
        def _(second_barrier):
            for nbr in (left, right):
                pl.semaphore_signal(second_barrier, inc=1, device_id=(nbr,),
                                    device_id_type=pl.DeviceIdType.MESH)
            pl.semaphore_wait(second_barrier, 2)

    return pl.pallas_call(
        body,
        out_shape=jax.ShapeDtypeStruct((tokens, h), jnp.float32),
        in_specs=[pl.BlockSpec(memory_space=pltpu.VMEM)] * 3
        + [pl.BlockSpec(memory_space=pltpu.SMEM)],
        out_specs=pl.BlockSpec(memory_space=pltpu.VMEM),
        scratch_shapes=[
            pltpu.VMEM((SLOTS_F, e_loc, d, h), jnp.bfloat16),
            pltpu.VMEM((SLOTS_B, e_loc, d, h), jnp.bfloat16),
            pltpu.VMEM((F_HOPS + 1, 8, N_EXPERTS), jnp.int32),
            pltpu.VMEM((B_HOPS + 1, 8, N_EXPERTS), jnp.int32),
            pltpu.SemaphoreType.DMA((F_HOPS * CHUNKS,)),
            pltpu.SemaphoreType.DMA((F_HOPS * CHUNKS,)),
            pltpu.SemaphoreType.DMA((B_HOPS * CHUNKS,)),
            pltpu.SemaphoreType.DMA((B_HOPS * CHUNKS,)),
            pltpu.SemaphoreType.DMA((F_HOPS,)),
            pltpu.SemaphoreType.DMA((F_HOPS,)),
            pltpu.SemaphoreType.DMA((B_HOPS,)),
            pltpu.SemaphoreType.DMA((B_HOPS,)),
        ],
        compiler_params=pltpu.CompilerParams(collective_id=0),
    )(x, route_idx, expert_W, meta)
